# baseline (speedup 1.0000x reference)
import functools

import jax
import jax.numpy as jnp
from jax import lax
from jax.experimental import pallas as pl
from jax.experimental.pallas import tpu as pltpu
from jax.experimental.pallas import tpu_sc as plsc

_B, _C, _N, _K = 4, 384, 1024, 20
_EPS = 1e-5
_NC, _NS, _L = 2, 16, 16
_NCG, _NPG = 8, 4
_CG = _C // _NCG
_PG = _N // _NPG
_GPW = _PG // _L


def _tc1_body(x_ref, we_ref, at_ref, bft_ref, idx_ref, mask_ref):
    xb = x_ref[0]
    w1 = we_ref[:, :_C]
    wd = we_ref[:, _C:] - w1
    at_ref[...] = jnp.dot(w1, xb, preferred_element_type=jnp.float32)
    bft_ref[...] = jnp.dot(wd, xb, preferred_element_type=jnp.float32)
    gram = lax.dot_general(xb, xb, (((0,), (0,)), ((), ())),
                           preferred_element_type=jnp.float32)
    xx = jnp.sum(xb * xb, axis=0)
    val = 2.0 * gram - xx[:, None] - xx[None, :]
    iota = lax.broadcasted_iota(jnp.int32, (_N, _N), 1)
    neg = jnp.float32(-jnp.inf)

    rm = jnp.max(val, axis=1, keepdims=True)
    for t in range(_K):
        eq = val == rm
        am = jnp.min(jnp.where(eq, iota, _N), axis=1, keepdims=True)
        idx_ref[:, t:t + 1] = am
        val = jnp.where(eq, neg, val)
        rm = jnp.max(val, axis=1, keepdims=True)
    mask_ref[...] = jnp.where(val == neg, 1.0, 0.0)


def _make_tc1(b):
    return pl.pallas_call(
        _tc1_body,
        grid=(1,),
        in_specs=[
            pl.BlockSpec((1, _C, _N), lambda i, _b=b: (_b, 0, 0)),
            pl.BlockSpec((_C, 2 * _C), lambda i: (0, 0)),
        ],
        out_specs=[
            pl.BlockSpec((_C, _N), lambda i: (0, 0)),
            pl.BlockSpec((_C, _N), lambda i: (0, 0)),
            pl.BlockSpec((_N, _K), lambda i: (0, 0)),
            pl.BlockSpec((_N, _N), lambda i: (0, 0)),
        ],
        out_shape=[
            jax.ShapeDtypeStruct((_C, _N), jnp.float32),
            jax.ShapeDtypeStruct((_C, _N), jnp.float32),
            jax.ShapeDtypeStruct((_N, _K), jnp.int32),
            jax.ShapeDtypeStruct((_N, _N), jnp.float32),
        ],
    )


_tc1s = [_make_tc1(b) for b in range(_B)]


def _sc_gather_body(at_hbm, idx_hbm, m_hbm, at_v, idx_v, m_v):
    wid = lax.axis_index("s") * _NC + lax.axis_index("c")
    cg = wid // _NPG
    pg = wid - cg * _NPG
    c_off = pl.multiple_of(cg * _CG, 8)
    p_off = pl.multiple_of(pg * _PG, 256)
    pltpu.sync_copy(at_hbm.at[pl.ds(c_off, _CG)], at_v)
    pltpu.sync_copy(idx_hbm.at[pl.ds(p_off, _PG)], idx_v)
    lanes = lax.iota(jnp.int32, _L)

    def group(g, carry):
        base = g * _L
        rows = base + lanes
        idxs = [plsc.load_gather(idx_v, [rows, jnp.full((_L,), k, jnp.int32)])
                for k in range(_K)]

        def chan(cp, carry2):
            for half in range(2):
                ci = cp * 2 + half
                row = jnp.full((_L,), ci, jnp.int32)
                m = plsc.load_gather(at_v, [row, idxs[0]])
                for k in range(1, _K):
                    m = jnp.maximum(m, plsc.load_gather(at_v, [row, idxs[k]]))
                m_v[ci, pl.ds(base, _L)] = m
            return carry2

        lax.fori_loop(0, _CG // 2, chan, 0)
        return carry

    lax.fori_loop(0, _GPW, group, 0)
    pltpu.sync_copy(m_v, m_hbm.at[pl.ds(c_off, _CG), pl.ds(p_off, _PG)])


@functools.lru_cache(maxsize=1)
def _get_sc_gather():
    mesh = plsc.VectorSubcoreMesh(core_axis_name="c", subcore_axis_name="s")
    return pl.kernel(
        _sc_gather_body,
        out_type=jax.ShapeDtypeStruct((_C, _N), jnp.float32),
        mesh=mesh,
        compiler_params=pltpu.CompilerParams(needs_layout_passes=False,
                                             use_tc_tiling_on_sc=True),
        scratch_types=[
            pltpu.VMEM((_CG, _N), jnp.float32),
            pltpu.VMEM((_PG, _K), jnp.int32),
            pltpu.VMEM((_CG, _PG), jnp.float32),
        ],
    )


def _tc1b_body(*refs):
    at_refs = refs[0:_B]
    bft_refs = refs[_B:2 * _B]
    mask_refs = refs[2 * _B:3 * _B]
    sat_refs = refs[3 * _B:4 * _B]
    sqat_refs = refs[4 * _B:5 * _B]
    cross_refs = refs[5 * _B:6 * _B]
    for b in range(_B):
        at = at_refs[b][...]
        mask = mask_refs[b][...]
        bm = lax.dot_general(bft_refs[b][...], mask, (((1,), (0,)), ((), ())),
                             preferred_element_type=jnp.float32)
        cnt = jnp.sum(mask, axis=0, keepdims=True)
        sat_refs[b][...] = jnp.sum(at * cnt, axis=1, keepdims=True)
        sqat_refs[b][...] = jnp.sum(at * at * cnt, axis=1, keepdims=True)
        cross_refs[b][...] = jnp.sum(at * bm, axis=1, keepdims=True)


_tc1b = pl.pallas_call(
    _tc1b_body,
    out_shape=[jax.ShapeDtypeStruct((_C, 1), jnp.float32)] * (3 * _B),
)


def _tc2_body(*refs):
    m_refs = refs[0:_B]
    bft_refs = refs[_B:2 * _B]
    sat_refs = refs[2 * _B:3 * _B]
    sqat_refs = refs[3 * _B:4 * _B]
    cross_refs = refs[4 * _B:5 * _B]
    wp_ref, ge_ref, be_ref, gp_ref, bp_ref, out_ref = refs[5 * _B:]

    cnt1 = float(_B * _N * _K)
    tsum = jnp.zeros((_C, 1), jnp.float32)
    tsq = jnp.zeros((_C, 1), jnp.float32)
    for b in range(_B):
        bf = bft_refs[b][...]
        tsum = tsum + sat_refs[b][...] + _K * jnp.sum(bf, axis=1,
                                                      keepdims=True)
        tsq = tsq + (sqat_refs[b][...] + 2.0 * cross_refs[b][...]
                     + _K * jnp.sum(bf * bf, axis=1, keepdims=True))
    mean1 = tsum / cnt1
    var1 = tsq / cnt1 - mean1 * mean1
    sc1 = ge_ref[...] * lax.rsqrt(var1 + _EPS)
    sh1 = be_ref[...] - mean1 * sc1

    ysum = jnp.zeros((_C, 1), jnp.float32)
    ysq = jnp.zeros((_C, 1), jnp.float32)
    ymaxes = []
    for b in range(_B):
        h = (m_refs[b][...] + bft_refs[b][...]) * sc1 + sh1
        h = jnp.where(h >= 0, h, 0.2 * h)
        y = jnp.dot(wp_ref[...], h, preferred_element_type=jnp.float32)
        ysum = ysum + jnp.sum(y, axis=1, keepdims=True)
        ysq = ysq + jnp.sum(y * y, axis=1, keepdims=True)
        ymaxes.append(jnp.max(y, axis=1, keepdims=True))
    cnt2 = float(_B * _N)
    mean2 = ysum / cnt2
    var2 = ysq / cnt2 - mean2 * mean2
    sc2 = gp_ref[...] * lax.rsqrt(var2 + _EPS)
    sh2 = bp_ref[...] - mean2 * sc2
    for b in range(_B):
        o = ymaxes[b] * sc2 + sh2
        out_ref[b] = jnp.where(o >= 0, o, 0.2 * o)


_tc2 = pl.pallas_call(
    _tc2_body,
    out_shape=jax.ShapeDtypeStruct((_B, _C, 1), jnp.float32),
)


def kernel(x, pos, W_e, g_e, b_e, W_p, g_p, b_p):
    sc_gather = _get_sc_gather()
    ms, ats, bfts, masks = [], [], [], []
    for b in range(_B):
        at_b, bft_b, idx_b, mask_b = _tc1s[b](x, W_e)
        ms.append(sc_gather(at_b, idx_b))
        ats.append(at_b)
        bfts.append(bft_b)
        masks.append(mask_b)
    stats = _tc1b(*ats, *bfts, *masks)
    sats, sqats, crosses = stats[:_B], stats[_B:2 * _B], stats[2 * _B:]
    out = _tc2(*ms, *bfts, *sats, *sqats, *crosses, W_p,
               g_e.reshape(_C, 1), b_e.reshape(_C, 1),
               g_p.reshape(_C, 1), b_p.reshape(_C, 1))
    return jnp.swapaxes(out, 1, 2)


# speedup vs baseline: 1.3764x; 1.3764x over previous
import functools

import jax
import jax.numpy as jnp
from jax import lax
from jax.experimental import pallas as pl
from jax.experimental.pallas import tpu as pltpu
from jax.experimental.pallas import tpu_sc as plsc

_B, _C, _N, _K = 4, 384, 1024, 20
_EPS = 1e-5
_NC, _NS, _L = 2, 16, 16
_NCG, _NPG = 8, 4
_CG = _C // _NCG
_PG = _N // _NPG
_GPW = _PG // _L


def _tc1_body(x_ref, we_ref, at_ref, bft_ref, idx_ref, mask_ref):
    xb = x_ref[0]
    w1 = we_ref[:, :_C]
    wd = we_ref[:, _C:] - w1
    at_ref[...] = jnp.dot(w1, xb, preferred_element_type=jnp.float32)
    bft_ref[...] = jnp.dot(wd, xb, preferred_element_type=jnp.float32)
    gram = lax.dot_general(xb, xb, (((0,), (0,)), ((), ())),
                           preferred_element_type=jnp.float32)
    xx = jnp.sum(xb * xb, axis=0)
    val = 2.0 * gram - xx[:, None] - xx[None, :]
    iota = lax.broadcasted_iota(jnp.int32, (_N, _N), 1)
    neg = jnp.float32(-jnp.inf)

    rm = jnp.max(val, axis=1, keepdims=True)
    for t in range(_K):
        eq = val == rm
        am = jnp.min(jnp.where(eq, iota, _N), axis=1, keepdims=True)
        idx_ref[:, t:t + 1] = am
        val = jnp.where(eq, neg, val)
        rm = jnp.max(val, axis=1, keepdims=True)
    mask_ref[...] = jnp.where(val == neg, 1.0, 0.0)


def _make_tc1(b):
    return pl.pallas_call(
        _tc1_body,
        grid=(1,),
        in_specs=[
            pl.BlockSpec((1, _C, _N), lambda i, _b=b: (_b, 0, 0)),
            pl.BlockSpec((_C, 2 * _C), lambda i: (0, 0)),
        ],
        out_specs=[
            pl.BlockSpec((_C, _N), lambda i: (0, 0)),
            pl.BlockSpec((_C, _N), lambda i: (0, 0)),
            pl.BlockSpec((_N, _K), lambda i: (0, 0)),
            pl.BlockSpec((_N, _N), lambda i: (0, 0)),
        ],
        out_shape=[
            jax.ShapeDtypeStruct((_C, _N), jnp.float32),
            jax.ShapeDtypeStruct((_C, _N), jnp.float32),
            jax.ShapeDtypeStruct((_N, _K), jnp.int32),
            jax.ShapeDtypeStruct((_N, _N), jnp.float32),
        ],
    )


_tc1s = [_make_tc1(b) for b in range(_B)]


def _sc_gather_body(at_hbm, idx_hbm, m_hbm, at_v, idx_v, m_v):
    wid = lax.axis_index("s") * _NC + lax.axis_index("c")
    cg = wid // _NPG
    pg = wid - cg * _NPG
    c_off = pl.multiple_of(cg * _CG, 8)
    p_off = pl.multiple_of(pg * _PG, 256)
    pltpu.sync_copy(at_hbm.at[pl.ds(c_off, _CG)], at_v)
    pltpu.sync_copy(idx_hbm.at[pl.ds(p_off, _PG)], idx_v)
    lanes = lax.iota(jnp.int32, _L)

    def group(g, carry):
        base = g * _L
        rows = base + lanes
        idxs = [plsc.load_gather(idx_v, [rows, jnp.full((_L,), k, jnp.int32)])
                for k in range(_K)]

        def chan(ci, carry2):
            row = jnp.full((_L,), ci, jnp.int32)
            m = plsc.load_gather(at_v, [row, idxs[0]])
            for k in range(1, _K):
                m = jnp.maximum(m, plsc.load_gather(at_v, [row, idxs[k]]))
            m_v[ci, pl.ds(base, _L)] = m
            return carry2

        lax.fori_loop(0, _CG, chan, 0)
        return carry

    lax.fori_loop(0, _GPW, group, 0)
    pltpu.sync_copy(m_v, m_hbm.at[pl.ds(c_off, _CG), pl.ds(p_off, _PG)])


@functools.lru_cache(maxsize=1)
def _get_sc_gather():
    mesh = plsc.VectorSubcoreMesh(core_axis_name="c", subcore_axis_name="s")
    return pl.kernel(
        _sc_gather_body,
        out_type=jax.ShapeDtypeStruct((_C, _N), jnp.float32),
        mesh=mesh,
        compiler_params=pltpu.CompilerParams(needs_layout_passes=False,
                                             use_tc_tiling_on_sc=True),
        scratch_types=[
            pltpu.VMEM((_CG, _N), jnp.float32),
            pltpu.VMEM((_PG, _K), jnp.int32),
            pltpu.VMEM((_CG, _PG), jnp.float32),
        ],
    )


def _tc1b_body(*refs):
    at_refs = refs[0:_B]
    bft_refs = refs[_B:2 * _B]
    mask_refs = refs[2 * _B:3 * _B]
    sat_refs = refs[3 * _B:4 * _B]
    sqat_refs = refs[4 * _B:5 * _B]
    cross_refs = refs[5 * _B:6 * _B]
    for b in range(_B):
        at = at_refs[b][...]
        mask = mask_refs[b][...]
        bm = lax.dot_general(bft_refs[b][...], mask, (((1,), (0,)), ((), ())),
                             preferred_element_type=jnp.float32)
        cnt = jnp.sum(mask, axis=0, keepdims=True)
        sat_refs[b][...] = jnp.sum(at * cnt, axis=1, keepdims=True)
        sqat_refs[b][...] = jnp.sum(at * at * cnt, axis=1, keepdims=True)
        cross_refs[b][...] = jnp.sum(at * bm, axis=1, keepdims=True)


_tc1b = pl.pallas_call(
    _tc1b_body,
    out_shape=[jax.ShapeDtypeStruct((_C, 1), jnp.float32)] * (3 * _B),
)


def _tc2_body(*refs):
    m_refs = refs[0:_B]
    bft_refs = refs[_B:2 * _B]
    sat_refs = refs[2 * _B:3 * _B]
    sqat_refs = refs[3 * _B:4 * _B]
    cross_refs = refs[4 * _B:5 * _B]
    wp_ref, ge_ref, be_ref, gp_ref, bp_ref, out_ref = refs[5 * _B:]

    cnt1 = float(_B * _N * _K)
    tsum = jnp.zeros((_C, 1), jnp.float32)
    tsq = jnp.zeros((_C, 1), jnp.float32)
    for b in range(_B):
        bf = bft_refs[b][...]
        tsum = tsum + sat_refs[b][...] + _K * jnp.sum(bf, axis=1,
                                                      keepdims=True)
        tsq = tsq + (sqat_refs[b][...] + 2.0 * cross_refs[b][...]
                     + _K * jnp.sum(bf * bf, axis=1, keepdims=True))
    mean1 = tsum / cnt1
    var1 = tsq / cnt1 - mean1 * mean1
    sc1 = ge_ref[...] * lax.rsqrt(var1 + _EPS)
    sh1 = be_ref[...] - mean1 * sc1

    ysum = jnp.zeros((_C, 1), jnp.float32)
    ysq = jnp.zeros((_C, 1), jnp.float32)
    ymaxes = []
    for b in range(_B):
        h = (m_refs[b][...] + bft_refs[b][...]) * sc1 + sh1
        h = jnp.where(h >= 0, h, 0.2 * h)
        y = jnp.dot(wp_ref[...], h, preferred_element_type=jnp.float32)
        ysum = ysum + jnp.sum(y, axis=1, keepdims=True)
        ysq = ysq + jnp.sum(y * y, axis=1, keepdims=True)
        ymaxes.append(jnp.max(y, axis=1, keepdims=True))
    cnt2 = float(_B * _N)
    mean2 = ysum / cnt2
    var2 = ysq / cnt2 - mean2 * mean2
    sc2 = gp_ref[...] * lax.rsqrt(var2 + _EPS)
    sh2 = bp_ref[...] - mean2 * sc2
    for b in range(_B):
        o = ymaxes[b] * sc2 + sh2
        out_ref[b] = jnp.where(o >= 0, o, 0.2 * o)


_tc2 = pl.pallas_call(
    _tc2_body,
    out_shape=jax.ShapeDtypeStruct((_B, _C, 1), jnp.float32),
)


def kernel(x, pos, W_e, g_e, b_e, W_p, g_p, b_p):
    sc_gather = _get_sc_gather()
    ms, ats, bfts, masks = [], [], [], []
    for b in range(_B):
        at_b, bft_b, idx_b, mask_b = _tc1s[b](x, W_e)
        ms.append(sc_gather(at_b, idx_b))
        ats.append(at_b)
        bfts.append(bft_b)
        masks.append(mask_b)
    stats = _tc1b(*ats, *bfts, *masks)
    sats, sqats, crosses = stats[:_B], stats[_B:2 * _B], stats[2 * _B:]
    out = _tc2(*ms, *bfts, *sats, *sqats, *crosses, W_p,
               g_e.reshape(_C, 1), b_e.reshape(_C, 1),
               g_p.reshape(_C, 1), b_p.reshape(_C, 1))
    return jnp.swapaxes(out, 1, 2)
